# baseline (speedup 1.0000x reference)
import functools

import jax
import jax.numpy as jnp
import numpy as np
from jax import lax
from jax.experimental import pallas as pl
from jax.experimental.pallas import tpu as pltpu
from jax.experimental.pallas import tpu_sc as plsc

DIM_ = 128
B_ = 16384
NC_ = 2
NS_ = 16
NW_ = NC_ * NS_
NCHUNK_ = 4
CB_ = B_ // NCHUNK_
BPW_ = CB_ // NW_
BM_ = 1024
NB_ = CB_ // BM_

_PERM = np.empty((DIM_,), np.int64)
for _k in range(DIM_ // 32):
    for _j in range(16):
        _PERM[32 * _k + 2 * _j] = 32 * _k + _j
        _PERM[32 * _k + 2 * _j + 1] = 32 * _k + 16 + _j


def _preproj_body(ed_ref, et_ref, w_ref, b_ref, o1_ref, o2_ref):
    o1_ref[...] = (jnp.dot(ed_ref[...], w_ref[pl.ds(DIM_, DIM_), :],
                           preferred_element_type=jnp.float32)
                   + b_ref[...])
    o2_ref[...] = jnp.dot(et_ref[...], w_ref[pl.ds(2 * DIM_, DIM_), :],
                          preferred_element_type=jnp.float32)


def _preproj(e_dose, e_time, w, b2):
    n = e_dose.shape[0]
    sds = jax.ShapeDtypeStruct((n, DIM_), jnp.float32)
    return pl.pallas_call(
        _preproj_body,
        out_shape=[sds, sds],
    )(e_dose, e_time, w, b2)


_RND = jnp.uint32(0x8000)
_HI = jnp.uint32(0xFFFF0000)


def _pack_pair(au, bu):
    au = lax.shift_right_logical(au + _RND, jnp.uint32(16))
    bu = (bu + _RND) & _HI
    return au | bu


def _pack_rows(src, dst, nrows):
    def _row(r, carry):
        for k in range(DIM_ // 32):
            dst[r, pl.ds(k * 16, 16)] = _pack_pair(
                src[r, pl.ds(k * 32, 16)],
                src[r, pl.ds(k * 32 + 16, 16)])
        return carry

    lax.fori_loop(0, nrows, _row, 0, unroll=2)


def _add_rows(s1, s2, nrows):
    def _row(r, carry):
        for k in range(DIM_ // 16):
            plsc.addupdate(s1.at[r, pl.ds(k * 16, 16)],
                           s2[r, pl.ds(k * 16, 16)])
        return carry

    lax.fori_loop(0, nrows, _row, 0, unroll=2)


def _sc_gather_body(c, ic_hbm, id_hbm, it_hbm, ib_hbm, tc_hbm, tb_hbm,
                    p1_hbm, p2_hbm, obig_hbm, os_hbm,
                    idx_v, rows_v, s_v, pk_v,
                    isem, gsem, wsem):
    wid = lax.axis_index("s") * NC_ + lax.axis_index("c")
    base = wid * BPW_
    src = c * CB_ + base
    ics = [
        pltpu.async_copy(h.at[pl.ds(src, BPW_)], idx_v.at[t], isem)
        for t, h in enumerate((id_hbm, it_hbm, ic_hbm, ib_hbm))
    ]
    for ic in ics:
        ic.wait()
    g1 = pltpu.async_copy(p1_hbm.at[idx_v.at[0]], s_v.at[0], gsem)
    g2 = pltpu.async_copy(p2_hbm.at[idx_v.at[1]], s_v.at[1], gsem)
    g0 = pltpu.async_copy(tc_hbm.at[idx_v.at[2]], rows_v.at[0], gsem)
    g3 = pltpu.async_copy(tb_hbm.at[idx_v.at[3]], rows_v.at[1], gsem)
    g1.wait()
    g2.wait()
    _add_rows(s_v.at[0], s_v.at[1], BPW_)
    ws0 = pltpu.async_copy(s_v.at[0], os_hbm.at[pl.ds(base, BPW_)], wsem)
    g0.wait()
    _pack_rows(rows_v.at[0], pk_v.at[0], BPW_)
    w0 = pltpu.async_copy(pk_v.at[0], obig_hbm.at[0, pl.ds(base, BPW_)],
                          wsem)
    g3.wait()
    _pack_rows(rows_v.at[1], pk_v.at[1], BPW_)
    w1 = pltpu.async_copy(pk_v.at[1], obig_hbm.at[1, pl.ds(base, BPW_)],
                          wsem)
    ws0.wait()
    w0.wait()
    w1.wait()


def _make_gather(c):
    return pl.kernel(
        functools.partial(_sc_gather_body, c),
        out_type=[
            jax.ShapeDtypeStruct((2, CB_, DIM_ // 2), jnp.uint32),
            jax.ShapeDtypeStruct((CB_, DIM_), jnp.float32),
        ],
        mesh=plsc.VectorSubcoreMesh(core_axis_name="c",
                                    subcore_axis_name="s"),
        scratch_types=[
            pltpu.VMEM((4, BPW_), jnp.int32),
            pltpu.VMEM((2, BPW_, DIM_), jnp.uint32),
            pltpu.VMEM((2, BPW_, DIM_), jnp.float32),
            pltpu.VMEM((2, BPW_, DIM_ // 2), jnp.uint32),
            pltpu.SemaphoreType.DMA,
            pltpu.SemaphoreType.DMA,
            pltpu.SemaphoreType.DMA,
        ],
    )


_gathers = [_make_gather(c) for c in range(NCHUNK_)]


def _proj_compute(x_ref, s_ref, wp_ref):
    return (s_ref[...]
            + jnp.dot(x_ref[0], wp_ref[pl.ds(0, DIM_), :],
                      preferred_element_type=jnp.float32)
            + jnp.dot(x_ref[1], wp_ref[pl.ds(DIM_, DIM_), :],
                      preferred_element_type=jnp.float32))


def _proj_body(x_ref, s_ref, wp_ref, o_ref):
    o_ref[...] = _proj_compute(x_ref, s_ref, wp_ref)


def _proj_body_alias(x_ref, s_ref, wp_ref, buf_ref, o_ref):
    o_ref[...] = _proj_compute(x_ref, s_ref, wp_ref)


def _proj(c, x, s, wp2, buf):
    common = dict(
        grid=(NB_,),
        out_specs=pl.BlockSpec((BM_, DIM_), lambda i: (c * NB_ + i, 0)),
        out_shape=jax.ShapeDtypeStruct((B_, DIM_), jnp.float32),
    )
    in_specs = [
        pl.BlockSpec((2, BM_, DIM_), lambda i: (0, i, 0)),
        pl.BlockSpec((BM_, DIM_), lambda i: (i, 0)),
        pl.BlockSpec((2 * DIM_, DIM_), lambda i: (0, 0)),
    ]
    if buf is None:
        return pl.pallas_call(
            _proj_body,
            in_specs=in_specs,
            **common,
        )(x, s, wp2)
    return pl.pallas_call(
        _proj_body_alias,
        in_specs=in_specs + [pl.BlockSpec(memory_space=pl.ANY)],
        input_output_aliases={3: 0},
        **common,
    )(x, s, wp2, buf)


def _as_bf16(u32_arr):
    bf = lax.bitcast_convert_type(u32_arr, jnp.bfloat16)
    return bf.reshape(*u32_arr.shape[:-1], u32_arr.shape[-1] * 2)


def kernel(cell_type, dose, time, batch, E_cell_type, E_dose, E_time,
           E_batch, W, b):
    ic = cell_type.astype(jnp.int32)
    id_ = dose.astype(jnp.int32)
    it = time.astype(jnp.int32)
    ib = batch.astype(jnp.int32)
    b2 = b.reshape(1, DIM_)
    tcb = lax.bitcast_convert_type(E_cell_type, jnp.uint32)
    tbb = lax.bitcast_convert_type(E_batch, jnp.uint32)
    perm = jnp.asarray(_PERM)
    w4 = W.reshape(4, DIM_, DIM_)
    wp2 = jnp.concatenate([w4[0][perm], w4[3][perm]], axis=0)
    p1, p2 = _preproj(E_dose, E_time, W, b2)
    buf = None
    for c in range(NCHUNK_):
        xu, sf = _gathers[c](ic, id_, it, ib, tcb, tbb, p1, p2)
        buf = _proj(c, _as_bf16(xu), sf, wp2, buf)
    return buf


# speedup vs baseline: 2.7786x; 2.7786x over previous
import functools

import jax
import jax.numpy as jnp
from jax import lax
from jax.experimental import pallas as pl
from jax.experimental.pallas import tpu as pltpu
from jax.experimental.pallas import tpu_sc as plsc

DIM_ = 128
B_ = 16384
NC_ = 2
NS_ = 16
NW_ = NC_ * NS_
NCHUNK_ = 4
CB_ = B_ // NCHUNK_
BPW_ = CB_ // NW_
BM_ = 1024
NB_ = CB_ // BM_


def _preproj_body(ed_ref, et_ref, w_ref, b_ref, o1_ref, o2_ref):
    o1_ref[...] = (jnp.dot(ed_ref[...], w_ref[pl.ds(DIM_, DIM_), :],
                           preferred_element_type=jnp.float32)
                   + b_ref[...])
    o2_ref[...] = jnp.dot(et_ref[...], w_ref[pl.ds(2 * DIM_, DIM_), :],
                          preferred_element_type=jnp.float32)


def _preproj(e_dose, e_time, w, b2):
    n = e_dose.shape[0]
    sds = jax.ShapeDtypeStruct((n, DIM_), jnp.float32)
    return pl.pallas_call(
        _preproj_body,
        out_shape=[sds, sds],
    )(e_dose, e_time, w, b2)


def _sc_gather_body(c, ic_hbm, id_hbm, it_hbm, ib_hbm, tc_hbm, tb_hbm,
                    p1_hbm, p2_hbm, out_hbm, idx_v, rows_v, s1_v, s2_v,
                    isem, gsem, wsem):
    wid = lax.axis_index("s") * NC_ + lax.axis_index("c")
    base = wid * BPW_
    src = c * CB_ + base
    ics = [
        pltpu.async_copy(h.at[pl.ds(src, BPW_)], idx_v.at[t], isem)
        for t, h in enumerate((id_hbm, it_hbm, ic_hbm, ib_hbm))
    ]
    for ic in ics:
        ic.wait()
    g1 = pltpu.async_copy(p1_hbm.at[idx_v.at[0]], s1_v, gsem)
    g2 = pltpu.async_copy(p2_hbm.at[idx_v.at[1]], s2_v, gsem)
    g0 = pltpu.async_copy(tc_hbm.at[idx_v.at[2]], rows_v.at[0], gsem)
    g3 = pltpu.async_copy(tb_hbm.at[idx_v.at[3]], rows_v.at[1], gsem)
    g1.wait()
    g2.wait()

    def _add_row(r, carry):
        for k in range(DIM_ // 16):
            plsc.addupdate(s1_v.at[r, pl.ds(k * 16, 16)],
                           s2_v[r, pl.ds(k * 16, 16)])
        return carry

    lax.fori_loop(0, BPW_, _add_row, 0, unroll=2)
    ws = pltpu.async_copy(s1_v, out_hbm.at[2, pl.ds(base, BPW_)], wsem)
    g0.wait()
    w0 = pltpu.async_copy(rows_v.at[0], out_hbm.at[0, pl.ds(base, BPW_)],
                          wsem)
    g3.wait()
    w1 = pltpu.async_copy(rows_v.at[1], out_hbm.at[1, pl.ds(base, BPW_)],
                          wsem)
    ws.wait()
    w0.wait()
    w1.wait()


def _make_gather(c):
    return pl.kernel(
        functools.partial(_sc_gather_body, c),
        out_type=jax.ShapeDtypeStruct((3, CB_, DIM_), jnp.float32),
        mesh=plsc.VectorSubcoreMesh(core_axis_name="c",
                                    subcore_axis_name="s"),
        scratch_types=[
            pltpu.VMEM((4, BPW_), jnp.int32),
            pltpu.VMEM((2, BPW_, DIM_), jnp.float32),
            pltpu.VMEM((BPW_, DIM_), jnp.float32),
            pltpu.VMEM((BPW_, DIM_), jnp.float32),
            pltpu.SemaphoreType.DMA,
            pltpu.SemaphoreType.DMA,
            pltpu.SemaphoreType.DMA,
        ],
    )


_gathers = [_make_gather(c) for c in range(NCHUNK_)]


def _proj_compute(x_ref, w_ref):
    return (x_ref[2]
            + jnp.dot(x_ref[0], w_ref[pl.ds(0, DIM_), :],
                      preferred_element_type=jnp.float32)
            + jnp.dot(x_ref[1], w_ref[pl.ds(3 * DIM_, DIM_), :],
                      preferred_element_type=jnp.float32))


def _proj_body(x_ref, w_ref, o_ref):
    o_ref[...] = _proj_compute(x_ref, w_ref)


def _proj_body_alias(x_ref, w_ref, buf_ref, o_ref):
    o_ref[...] = _proj_compute(x_ref, w_ref)


def _proj(c, x, w, buf):
    common = dict(
        grid=(NB_,),
        out_specs=pl.BlockSpec((BM_, DIM_), lambda i: (c * NB_ + i, 0)),
        out_shape=jax.ShapeDtypeStruct((B_, DIM_), jnp.float32),
    )
    x_spec = pl.BlockSpec((3, BM_, DIM_), lambda i: (0, i, 0))
    w_spec = pl.BlockSpec((4 * DIM_, DIM_), lambda i: (0, 0))
    if buf is None:
        return pl.pallas_call(
            _proj_body,
            in_specs=[x_spec, w_spec],
            **common,
        )(x, w)
    return pl.pallas_call(
        _proj_body_alias,
        in_specs=[x_spec, w_spec, pl.BlockSpec(memory_space=pl.ANY)],
        input_output_aliases={2: 0},
        **common,
    )(x, w, buf)


def kernel(cell_type, dose, time, batch, E_cell_type, E_dose, E_time,
           E_batch, W, b):
    ic = cell_type.astype(jnp.int32)
    id_ = dose.astype(jnp.int32)
    it = time.astype(jnp.int32)
    ib = batch.astype(jnp.int32)
    p1, p2 = _preproj(E_dose, E_time, W, b.reshape(1, DIM_))
    buf = None
    for c in range(NCHUNK_):
        gathered = _gathers[c](ic, id_, it, ib, E_cell_type, E_batch,
                               p1, p2)
        buf = _proj(c, gathered, W, buf)
    return buf
